# baseline (speedup 1.0000x reference)
import functools

import jax
import jax.numpy as jnp
from jax import lax
from jax.experimental import pallas as pl
from jax.experimental.pallas import tpu as pltpu
from jax.experimental.pallas import tpu_sc as plsc

N = 10000
D = 128
E = 320000
NC = 2
NS = 16
NW = NC * NS
CH = 128
S = 2
R = 3
K = 80
E_PAD = NW * K * CH
N_PAD = 10112
ROWS_PT = N_PAD // NS
ZCH = 96


def _sc_cnt_body(dst_hbm, cnt_out, dst_v, ones_v, zrow_v, cnt_sh, sem):
    c = lax.axis_index("c")
    s = lax.axis_index("s")
    wid = c * NS + s
    zero16 = jnp.zeros((16,), jnp.float32)
    one16 = jnp.ones((16,), jnp.float32)

    def zrow_body(i, carry):
        for j in range(D // 16):
            zrow_v[i, pl.ds(j * 16, 16)] = zero16
        return carry

    lax.fori_loop(0, ZCH, zrow_body, 0)

    def ones_body(i, carry):
        for j in range(D // 16):
            ones_v[i, pl.ds(j * 16, 16)] = one16
        return carry

    lax.fori_loop(0, CH, ones_body, 0)

    base = s * ROWS_PT

    for z in range(ROWS_PT // ZCH):
        pltpu.sync_copy(zrow_v, cnt_sh.at[pl.ds(base + z * ZCH, ZCH)])
    rem = ROWS_PT - (ROWS_PT // ZCH) * ZCH
    if rem:
        pltpu.sync_copy(zrow_v.at[pl.ds(0, rem)],
                        cnt_sh.at[pl.ds(base + ROWS_PT - rem, rem)])
    pltpu.sync_copy(dst_hbm.at[wid], dst_v)
    plsc.subcore_barrier()

    def chunk_body(j, carry):
        pltpu.sync_copy(ones_v, cnt_sh.at[dst_v.at[j]], add=True)
        return carry

    lax.fori_loop(0, K, chunk_body, 0)
    plsc.subcore_barrier()

    pltpu.sync_copy(cnt_sh.at[pl.ds(base, ROWS_PT)],
                    cnt_out.at[c, pl.ds(base, ROWS_PT)])


@functools.cache
def _sc_cnt():
    mesh = plsc.VectorSubcoreMesh(core_axis_name="c", subcore_axis_name="s",
                                  num_cores=NC, num_subcores=NS)
    return pl.kernel(
        _sc_cnt_body,
        out_type=(jax.ShapeDtypeStruct((NC, N_PAD, D), jnp.float32),),
        mesh=mesh,
        scratch_types=[
            pltpu.VMEM((K, CH), jnp.int32),
            pltpu.VMEM((CH, D), jnp.float32),
            pltpu.VMEM((ZCH, D), jnp.float32),
            pltpu.VMEM_SHARED((N_PAD, D), jnp.float32),
            pltpu.SemaphoreType.DMA,
        ],
    )


def _sc_body(h_hbm, src_hbm, dst_hbm, parts_out,
             src_v, dst_v, rows_v, zrow_v, acc_sh, sem):
    c = lax.axis_index("c")
    s = lax.axis_index("s")
    wid = c * NS + s
    zero16 = jnp.zeros((16,), jnp.float32)

    def zrow_body(i, carry):
        for j in range(D // 16):
            zrow_v[i, pl.ds(j * 16, 16)] = zero16
        return carry

    lax.fori_loop(0, ZCH, zrow_body, 0)

    base = s * ROWS_PT

    for z in range(ROWS_PT // ZCH):
        pltpu.sync_copy(zrow_v, acc_sh.at[pl.ds(base + z * ZCH, ZCH)])
    rem = ROWS_PT - (ROWS_PT // ZCH) * ZCH
    if rem:
        pltpu.sync_copy(zrow_v.at[pl.ds(0, rem)],
                        acc_sh.at[pl.ds(base + ROWS_PT - rem, rem)])

    pltpu.sync_copy(src_hbm.at[wid], src_v)
    pltpu.sync_copy(dst_hbm.at[wid], dst_v)
    plsc.subcore_barrier()

    def chunk_body(j, carry):
        pltpu.async_copy(h_hbm.at[src_v.at[j]], rows_v, sem).wait()
        pltpu.sync_copy(rows_v, acc_sh.at[dst_v.at[j]], add=True)
        return carry

    lax.fori_loop(0, K, chunk_body, 0)
    plsc.subcore_barrier()

    pltpu.sync_copy(acc_sh.at[pl.ds(base, ROWS_PT)],
                    parts_out.at[c, pl.ds(base, ROWS_PT)])


@functools.cache
def _sc_segsum():
    mesh = plsc.VectorSubcoreMesh(core_axis_name="c", subcore_axis_name="s",
                                  num_cores=NC, num_subcores=NS)
    return pl.kernel(
        _sc_body,
        out_type=(jax.ShapeDtypeStruct((NC, N_PAD, D), jnp.float32),),
        mesh=mesh,
        scratch_types=[
            pltpu.VMEM((K, CH), jnp.int32),
            pltpu.VMEM((K, CH), jnp.int32),
            pltpu.VMEM((CH, D), jnp.float32),
            pltpu.VMEM((ZCH, D), jnp.float32),
            pltpu.VMEM_SHARED((N_PAD, D), jnp.float32),
            pltpu.SemaphoreType.DMA,
        ],
    )


_BLK = 2000


def _pre_body(x_ref, w_ref, b_ref, o_ref):
    o_ref[...] = lax.dot_general(
        x_ref[...], w_ref[...], (((1,), (1,)), ((), ())),
        preferred_element_type=jnp.float32) + b_ref[...]


_tc_pre = pl.pallas_call(
    _pre_body,
    grid=(N // _BLK,),
    in_specs=[
        pl.BlockSpec((_BLK, D), lambda i: (i, 0)),
        pl.BlockSpec((D, D), lambda i: (0, 0)),
        pl.BlockSpec((1, D), lambda i: (0, 0)),
    ],
    out_specs=pl.BlockSpec((_BLK, D), lambda i: (i, 0)),
    out_shape=jax.ShapeDtypeStruct((N, D), jnp.float32),
)


def _layer_body(p0_ref, p1_ref, c0_ref, c1_ref, h_ref, wl_ref, bl_ref, wr_ref, o_ref):
    cnt = c0_ref[:, 0:1] + c1_ref[:, 0:1]
    agg = (p0_ref[...] + p1_ref[...]) / jnp.maximum(cnt, 1.0)
    y = lax.dot_general(agg, wl_ref[...], (((1,), (1,)), ((), ())),
                        preferred_element_type=jnp.float32)
    y = y + lax.dot_general(h_ref[...], wr_ref[...], (((1,), (1,)), ((), ())),
                            preferred_element_type=jnp.float32)
    o_ref[...] = jnp.maximum(y + bl_ref[...], 0.0)


_tc_layer = pl.pallas_call(
    _layer_body,
    grid=(N // _BLK,),
    in_specs=[
        pl.BlockSpec((_BLK, D), lambda i: (i, 0)),
        pl.BlockSpec((_BLK, D), lambda i: (i, 0)),
        pl.BlockSpec((_BLK, D), lambda i: (i, 0)),
        pl.BlockSpec((_BLK, D), lambda i: (i, 0)),
        pl.BlockSpec((_BLK, D), lambda i: (i, 0)),
        pl.BlockSpec((D, D), lambda i: (0, 0)),
        pl.BlockSpec((1, D), lambda i: (0, 0)),
        pl.BlockSpec((D, D), lambda i: (0, 0)),
    ],
    out_specs=pl.BlockSpec((_BLK, D), lambda i: (i, 0)),
    out_shape=jax.ShapeDtypeStruct((N, D), jnp.float32),
)


def _final_body(p0_ref, p1_ref, c0_ref, c1_ref, h_ref, wl_ref, bl_ref, wr_ref,
                wp1_ref, bp1_ref, wp2_ref, bp2_ref, o_ref):
    cnt = c0_ref[:, 0:1] + c1_ref[:, 0:1]
    agg = (p0_ref[...] + p1_ref[...]) / jnp.maximum(cnt, 1.0)
    y = lax.dot_general(agg, wl_ref[...], (((1,), (1,)), ((), ())),
                        preferred_element_type=jnp.float32)
    y = y + lax.dot_general(h_ref[...], wr_ref[...], (((1,), (1,)), ((), ())),
                            preferred_element_type=jnp.float32)
    y = jnp.maximum(y + bl_ref[...], 0.0)
    y = jnp.maximum(
        lax.dot_general(y, wp1_ref[...], (((1,), (1,)), ((), ())),
                        preferred_element_type=jnp.float32) + bp1_ref[...], 0.0)
    o_ref[...] = lax.dot_general(
        y, wp2_ref[...], (((1,), (1,)), ((), ())),
        preferred_element_type=jnp.float32) + bp2_ref[...]


_tc_final = pl.pallas_call(
    _final_body,
    grid=(N // _BLK,),
    in_specs=[
        pl.BlockSpec((_BLK, D), lambda i: (i, 0)),
        pl.BlockSpec((_BLK, D), lambda i: (i, 0)),
        pl.BlockSpec((_BLK, D), lambda i: (i, 0)),
        pl.BlockSpec((_BLK, D), lambda i: (i, 0)),
        pl.BlockSpec((_BLK, D), lambda i: (i, 0)),
        pl.BlockSpec((D, D), lambda i: (0, 0)),
        pl.BlockSpec((1, D), lambda i: (0, 0)),
        pl.BlockSpec((D, D), lambda i: (0, 0)),
        pl.BlockSpec((D, D), lambda i: (0, 0)),
        pl.BlockSpec((1, D), lambda i: (0, 0)),
        pl.BlockSpec((D, D), lambda i: (0, 0)),
        pl.BlockSpec((1, D), lambda i: (0, 0)),
    ],
    out_specs=pl.BlockSpec((_BLK, D), lambda i: (i, 0)),
    out_shape=jax.ShapeDtypeStruct((N, D), jnp.float32),
)


def kernel(x, edge_attr, edge_index, W_pre, b_pre, Wl1, bl1, Wr1, We1, be1,
           Wl2, bl2, Wr2, We2, be2, Wp1, bp1, Wp2, bp2):
    del edge_attr, We1, be1, We2, be2
    src = edge_index[0]
    dst = edge_index[1]
    pad = E_PAD - E
    srcp = jnp.concatenate([src, jnp.zeros((pad,), jnp.int32)]).reshape(NW, K, CH)
    dst_fill = N + (jnp.arange(pad, dtype=jnp.int32) % (N_PAD - N))
    dstp = jnp.concatenate([dst, dst_fill]).reshape(NW, K, CH)

    b_pre2 = b_pre.reshape(1, D)
    bl1_2 = bl1.reshape(1, D)
    bl2_2 = bl2.reshape(1, D)
    bp1_2 = bp1.reshape(1, D)
    bp2_2 = bp2.reshape(1, D)

    h0 = _tc_pre(x, W_pre, b_pre2)
    (cnts1,) = _sc_cnt()(dstp)
    h0b, cdep = lax.optimization_barrier((h0, cnts1))
    (parts1,) = _sc_segsum()(h0b, srcp, dstp)
    cnts1 = cdep
    c0 = cnts1[0, :N]
    c1 = cnts1[1, :N]
    h1 = _tc_layer(parts1[0, :N], parts1[1, :N], c0, c1, h0, Wl1, bl1_2, Wr1)
    (parts2,) = _sc_segsum()(h1, srcp, dstp)
    out = _tc_final(parts2[0, :N], parts2[1, :N], c0, c1, h1,
                    Wl2, bl2_2, Wr2, Wp1, bp1_2, Wp2, bp2_2)
    return out


# speedup vs baseline: 1.4907x; 1.4907x over previous
import functools

import jax
import jax.numpy as jnp
from jax import lax
from jax.experimental import pallas as pl
from jax.experimental.pallas import tpu as pltpu
from jax.experimental.pallas import tpu_sc as plsc

N = 10000
D = 128
E = 320000
NC = 2
NS = 16
NW = NC * NS
CH = 128
S = 2
R = 3
K = 79
E_PAD = NW * K * CH
N_PAD = 10112
ROWS_PT = N_PAD // NS
ZCH = 104


def _sc_cnt_body(dst_hbm, cnt_out, dst_v, ones_v, zrow_v, cnt_sh, sem):
    c = lax.axis_index("c")
    s = lax.axis_index("s")
    wid = c * NS + s
    zero16 = jnp.zeros((16,), jnp.float32)
    one16 = jnp.ones((16,), jnp.float32)

    def zrow_body(i, carry):
        for j in range(D // 16):
            zrow_v[i, pl.ds(j * 16, 16)] = zero16
        return carry

    lax.fori_loop(0, ZCH, zrow_body, 0)

    def ones_body(i, carry):
        for j in range(D // 16):
            ones_v[i, pl.ds(j * 16, 16)] = one16
        return carry

    lax.fori_loop(0, CH, ones_body, 0)

    base = s * ROWS_PT

    for z in range(ROWS_PT // ZCH):
        pltpu.sync_copy(zrow_v, cnt_sh.at[pl.ds(base + z * ZCH, ZCH)])
    rem = ROWS_PT - (ROWS_PT // ZCH) * ZCH
    if rem:
        pltpu.sync_copy(zrow_v.at[pl.ds(0, rem)],
                        cnt_sh.at[pl.ds(base + ROWS_PT - rem, rem)])
    pltpu.sync_copy(dst_hbm.at[wid], dst_v)
    plsc.subcore_barrier()

    def chunk_body(j, carry):
        pltpu.sync_copy(ones_v, cnt_sh.at[dst_v.at[j]], add=True)
        return carry

    lax.fori_loop(0, K, chunk_body, 0)
    plsc.subcore_barrier()

    pltpu.sync_copy(cnt_sh.at[pl.ds(base, ROWS_PT)],
                    cnt_out.at[c, pl.ds(base, ROWS_PT)])


@functools.cache
def _sc_cnt():
    mesh = plsc.VectorSubcoreMesh(core_axis_name="c", subcore_axis_name="s",
                                  num_cores=NC, num_subcores=NS)
    return pl.kernel(
        _sc_cnt_body,
        out_type=(jax.ShapeDtypeStruct((NC, N_PAD, D), jnp.float32),),
        mesh=mesh,
        scratch_types=[
            pltpu.VMEM((K, CH), jnp.int32),
            pltpu.VMEM((CH, D), jnp.float32),
            pltpu.VMEM((ZCH, D), jnp.float32),
            pltpu.VMEM_SHARED((N_PAD, D), jnp.float32),
            pltpu.SemaphoreType.DMA,
        ],
    )


def _sc_body(h_hbm, src_hbm, dst_hbm, parts_out,
             src_v, dst_v, rows_v, zrow_v, acc_sh, sem):
    c = lax.axis_index("c")
    s = lax.axis_index("s")
    wid = c * NS + s
    zero16 = jnp.zeros((16,), jnp.float32)

    def zrow_body(i, carry):
        for j in range(D // 16):
            zrow_v[i, pl.ds(j * 16, 16)] = zero16
        return carry

    lax.fori_loop(0, ZCH, zrow_body, 0)

    base = s * ROWS_PT

    for z in range(ROWS_PT // ZCH):
        pltpu.sync_copy(zrow_v, acc_sh.at[pl.ds(base + z * ZCH, ZCH)])
    rem = ROWS_PT - (ROWS_PT // ZCH) * ZCH
    if rem:
        pltpu.sync_copy(zrow_v.at[pl.ds(0, rem)],
                        acc_sh.at[pl.ds(base + ROWS_PT - rem, rem)])

    pltpu.sync_copy(src_hbm.at[wid], src_v)
    pltpu.sync_copy(dst_hbm.at[wid], dst_v)
    plsc.subcore_barrier()

    def chunk_body(j, carry):
        pltpu.async_copy(h_hbm.at[src_v.at[j]], rows_v, sem).wait()
        pltpu.sync_copy(rows_v, acc_sh.at[dst_v.at[j]], add=True)
        return carry

    lax.fori_loop(0, K, chunk_body, 0)
    plsc.subcore_barrier()

    pltpu.sync_copy(acc_sh.at[pl.ds(base, ROWS_PT)],
                    parts_out.at[c, pl.ds(base, ROWS_PT)])


@functools.cache
def _sc_segsum():
    mesh = plsc.VectorSubcoreMesh(core_axis_name="c", subcore_axis_name="s",
                                  num_cores=NC, num_subcores=NS)
    return pl.kernel(
        _sc_body,
        out_type=(jax.ShapeDtypeStruct((NC, N_PAD, D), jnp.float32),),
        mesh=mesh,
        scratch_types=[
            pltpu.VMEM((K, CH), jnp.int32),
            pltpu.VMEM((K, CH), jnp.int32),
            pltpu.VMEM((CH, D), jnp.float32),
            pltpu.VMEM((ZCH, D), jnp.float32),
            pltpu.VMEM_SHARED((N_PAD, D), jnp.float32),
            pltpu.SemaphoreType.DMA,
        ],
    )


_BLK = 2000


def _pre_body(x_ref, w_ref, b_ref, o_ref):
    o_ref[...] = lax.dot_general(
        x_ref[...], w_ref[...], (((1,), (1,)), ((), ())),
        preferred_element_type=jnp.float32) + b_ref[...]


_tc_pre = pl.pallas_call(
    _pre_body,
    grid=(N // _BLK,),
    in_specs=[
        pl.BlockSpec((_BLK, D), lambda i: (i, 0)),
        pl.BlockSpec((D, D), lambda i: (0, 0)),
        pl.BlockSpec((1, D), lambda i: (0, 0)),
    ],
    out_specs=pl.BlockSpec((_BLK, D), lambda i: (i, 0)),
    out_shape=jax.ShapeDtypeStruct((N, D), jnp.float32),
)


def _layer_body(p0_ref, p1_ref, c0_ref, c1_ref, h_ref, wl_ref, bl_ref, wr_ref, o_ref):
    cnt = c0_ref[:, 0:1] + c1_ref[:, 0:1]
    agg = (p0_ref[...] + p1_ref[...]) / jnp.maximum(cnt, 1.0)
    y = lax.dot_general(agg, wl_ref[...], (((1,), (1,)), ((), ())),
                        preferred_element_type=jnp.float32)
    y = y + lax.dot_general(h_ref[...], wr_ref[...], (((1,), (1,)), ((), ())),
                            preferred_element_type=jnp.float32)
    o_ref[...] = jnp.maximum(y + bl_ref[...], 0.0)


_tc_layer = pl.pallas_call(
    _layer_body,
    grid=(N // _BLK,),
    in_specs=[
        pl.BlockSpec((_BLK, D), lambda i: (i, 0)),
        pl.BlockSpec((_BLK, D), lambda i: (i, 0)),
        pl.BlockSpec((_BLK, D), lambda i: (i, 0)),
        pl.BlockSpec((_BLK, D), lambda i: (i, 0)),
        pl.BlockSpec((_BLK, D), lambda i: (i, 0)),
        pl.BlockSpec((D, D), lambda i: (0, 0)),
        pl.BlockSpec((1, D), lambda i: (0, 0)),
        pl.BlockSpec((D, D), lambda i: (0, 0)),
    ],
    out_specs=pl.BlockSpec((_BLK, D), lambda i: (i, 0)),
    out_shape=jax.ShapeDtypeStruct((N, D), jnp.float32),
)


def _final_body(p0_ref, p1_ref, c0_ref, c1_ref, h_ref, wl_ref, bl_ref, wr_ref,
                wp1_ref, bp1_ref, wp2_ref, bp2_ref, o_ref):
    cnt = c0_ref[:, 0:1] + c1_ref[:, 0:1]
    agg = (p0_ref[...] + p1_ref[...]) / jnp.maximum(cnt, 1.0)
    y = lax.dot_general(agg, wl_ref[...], (((1,), (1,)), ((), ())),
                        preferred_element_type=jnp.float32)
    y = y + lax.dot_general(h_ref[...], wr_ref[...], (((1,), (1,)), ((), ())),
                            preferred_element_type=jnp.float32)
    y = jnp.maximum(y + bl_ref[...], 0.0)
    y = jnp.maximum(
        lax.dot_general(y, wp1_ref[...], (((1,), (1,)), ((), ())),
                        preferred_element_type=jnp.float32) + bp1_ref[...], 0.0)
    o_ref[...] = lax.dot_general(
        y, wp2_ref[...], (((1,), (1,)), ((), ())),
        preferred_element_type=jnp.float32) + bp2_ref[...]


_tc_final = pl.pallas_call(
    _final_body,
    grid=(N // _BLK,),
    in_specs=[
        pl.BlockSpec((_BLK, D), lambda i: (i, 0)),
        pl.BlockSpec((_BLK, D), lambda i: (i, 0)),
        pl.BlockSpec((_BLK, D), lambda i: (i, 0)),
        pl.BlockSpec((_BLK, D), lambda i: (i, 0)),
        pl.BlockSpec((_BLK, D), lambda i: (i, 0)),
        pl.BlockSpec((D, D), lambda i: (0, 0)),
        pl.BlockSpec((1, D), lambda i: (0, 0)),
        pl.BlockSpec((D, D), lambda i: (0, 0)),
        pl.BlockSpec((D, D), lambda i: (0, 0)),
        pl.BlockSpec((1, D), lambda i: (0, 0)),
        pl.BlockSpec((D, D), lambda i: (0, 0)),
        pl.BlockSpec((1, D), lambda i: (0, 0)),
    ],
    out_specs=pl.BlockSpec((_BLK, D), lambda i: (i, 0)),
    out_shape=jax.ShapeDtypeStruct((N, D), jnp.float32),
)


def kernel(x, edge_attr, edge_index, W_pre, b_pre, Wl1, bl1, Wr1, We1, be1,
           Wl2, bl2, Wr2, We2, be2, Wp1, bp1, Wp2, bp2):
    del edge_attr, We1, be1, We2, be2
    src = edge_index[0]
    dst = edge_index[1]
    pad = E_PAD - E
    srcp = jnp.concatenate([src, jnp.zeros((pad,), jnp.int32)]).reshape(NW, K, CH)
    dst_fill = N + (jnp.arange(pad, dtype=jnp.int32) % (N_PAD - N))
    dstp = jnp.concatenate([dst, dst_fill]).reshape(NW, K, CH)

    b_pre2 = b_pre.reshape(1, D)
    bl1_2 = bl1.reshape(1, D)
    bl2_2 = bl2.reshape(1, D)
    bp1_2 = bp1.reshape(1, D)
    bp2_2 = bp2.reshape(1, D)

    h0 = _tc_pre(x, W_pre, b_pre2)
    (cnts1,) = _sc_cnt()(dstp)
    h0b, cdep = lax.optimization_barrier((h0, cnts1))
    (parts1,) = _sc_segsum()(h0b, srcp, dstp)
    cnts1 = cdep
    c0 = cnts1[0, :N]
    c1 = cnts1[1, :N]
    h1 = _tc_layer(parts1[0, :N], parts1[1, :N], c0, c1, h0, Wl1, bl1_2, Wr1)
    (parts2,) = _sc_segsum()(h1, srcp, dstp)
    out = _tc_final(parts2[0, :N], parts2[1, :N], c0, c1, h1,
                    Wl2, bl2_2, Wr2, Wp1, bp1_2, Wp2, bp2_2)
    return out


# speedup vs baseline: 2.6739x; 1.7937x over previous
import functools

import jax
import jax.numpy as jnp
from jax import lax
from jax.experimental import pallas as pl
from jax.experimental.pallas import tpu as pltpu
from jax.experimental.pallas import tpu_sc as plsc

N = 10000
D = 128
E = 320000
NC = 2
NS = 16
NW = NC * NS
CH = 128
S = 2
R = 3
K = 79
E_PAD = NW * K * CH
N_PAD = 10112
ROWS_PT = N_PAD // NS
ZCH = 104


def _sc_cnt_body(dst_hbm, cnt_out, dst_v, ones_v, zrow_v, cnt_sh, sem):
    c = lax.axis_index("c")
    s = lax.axis_index("s")
    wid = c * NS + s
    zero16 = jnp.zeros((16,), jnp.float32)
    one16 = jnp.ones((16,), jnp.float32)

    def zrow_body(i, carry):
        for j in range(D // 16):
            zrow_v[i, pl.ds(j * 16, 16)] = zero16
        return carry

    lax.fori_loop(0, ZCH, zrow_body, 0)

    def ones_body(i, carry):
        for j in range(D // 16):
            ones_v[i, pl.ds(j * 16, 16)] = one16
        return carry

    lax.fori_loop(0, CH, ones_body, 0)

    base = s * ROWS_PT

    for z in range(ROWS_PT // ZCH):
        pltpu.sync_copy(zrow_v, cnt_sh.at[pl.ds(base + z * ZCH, ZCH)])
    rem = ROWS_PT - (ROWS_PT // ZCH) * ZCH
    if rem:
        pltpu.sync_copy(zrow_v.at[pl.ds(0, rem)],
                        cnt_sh.at[pl.ds(base + ROWS_PT - rem, rem)])
    pltpu.sync_copy(dst_hbm.at[wid], dst_v)
    plsc.subcore_barrier()

    def chunk_body(j, carry):
        pltpu.sync_copy(ones_v, cnt_sh.at[dst_v.at[j]], add=True)
        return carry

    lax.fori_loop(0, K, chunk_body, 0)
    plsc.subcore_barrier()

    pltpu.sync_copy(cnt_sh.at[pl.ds(base, ROWS_PT)],
                    cnt_out.at[c, pl.ds(base, ROWS_PT)])


@functools.cache
def _sc_cnt():
    mesh = plsc.VectorSubcoreMesh(core_axis_name="c", subcore_axis_name="s",
                                  num_cores=NC, num_subcores=NS)
    return pl.kernel(
        _sc_cnt_body,
        out_type=(jax.ShapeDtypeStruct((NC, N_PAD, D), jnp.float32),),
        mesh=mesh,
        scratch_types=[
            pltpu.VMEM((K, CH), jnp.int32),
            pltpu.VMEM((CH, D), jnp.float32),
            pltpu.VMEM((ZCH, D), jnp.float32),
            pltpu.VMEM_SHARED((N_PAD, D), jnp.float32),
            pltpu.SemaphoreType.DMA,
        ],
    )


def _sc_body(h_hbm, src_hbm, dst_hbm, parts_out,
             src_v, dst_v, rows_v, zrow_v, acc_sh, sem):
    c = lax.axis_index("c")
    s = lax.axis_index("s")
    wid = c * NS + s
    zero16 = jnp.zeros((16,), jnp.float32)

    def zrow_body(i, carry):
        for j in range(D // 16):
            zrow_v[i, pl.ds(j * 16, 16)] = zero16
        return carry

    lax.fori_loop(0, ZCH, zrow_body, 0)

    base = s * ROWS_PT

    for z in range(ROWS_PT // ZCH):
        pltpu.sync_copy(zrow_v, acc_sh.at[pl.ds(base + z * ZCH, ZCH)])
    rem = ROWS_PT - (ROWS_PT // ZCH) * ZCH
    if rem:
        pltpu.sync_copy(zrow_v.at[pl.ds(0, rem)],
                        acc_sh.at[pl.ds(base + ROWS_PT - rem, rem)])

    pltpu.sync_copy(src_hbm.at[wid], src_v)
    pltpu.sync_copy(dst_hbm.at[wid], dst_v)
    plsc.subcore_barrier()

    def chunk_body(j, carry):
        pltpu.async_copy(h_hbm.at[src_v.at[j]], rows_v, sem).wait()
        pltpu.sync_copy(rows_v, acc_sh.at[dst_v.at[j]], add=True)
        return carry

    lax.fori_loop(0, K, chunk_body, 0)
    plsc.subcore_barrier()

    pltpu.sync_copy(acc_sh.at[pl.ds(base, ROWS_PT)],
                    parts_out.at[c, pl.ds(base, ROWS_PT)])


@functools.cache
def _sc_segsum():
    mesh = plsc.VectorSubcoreMesh(core_axis_name="c", subcore_axis_name="s",
                                  num_cores=NC, num_subcores=NS)
    return pl.kernel(
        _sc_body,
        out_type=(jax.ShapeDtypeStruct((NC, N_PAD, D), jnp.float32),),
        mesh=mesh,
        scratch_types=[
            pltpu.VMEM((K, CH), jnp.int32),
            pltpu.VMEM((K, CH), jnp.int32),
            pltpu.VMEM((CH, D), jnp.float32),
            pltpu.VMEM((ZCH, D), jnp.float32),
            pltpu.VMEM_SHARED((N_PAD, D), jnp.float32),
            pltpu.SemaphoreType.DMA,
        ],
    )


_BLK = 2000


def _pre_body(x_ref, w_ref, b_ref, o_ref):
    o_ref[...] = lax.dot_general(
        x_ref[...], w_ref[...], (((1,), (1,)), ((), ())),
        preferred_element_type=jnp.float32) + b_ref[...]


_tc_pre = pl.pallas_call(
    _pre_body,
    grid=(N // _BLK,),
    in_specs=[
        pl.BlockSpec((_BLK, D), lambda i: (i, 0)),
        pl.BlockSpec((D, D), lambda i: (0, 0)),
        pl.BlockSpec((1, D), lambda i: (0, 0)),
    ],
    out_specs=pl.BlockSpec((_BLK, D), lambda i: (i, 0)),
    out_shape=jax.ShapeDtypeStruct((N, D), jnp.float32),
)


def _layer_body(p0_ref, p1_ref, c0_ref, c1_ref, h_ref, wl_ref, bl_ref, wr_ref, o_ref):
    cnt = c0_ref[:, 0:1] + c1_ref[:, 0:1]
    agg = (p0_ref[...] + p1_ref[...]) / jnp.maximum(cnt, 1.0)
    y = lax.dot_general(agg, wl_ref[...], (((1,), (1,)), ((), ())),
                        preferred_element_type=jnp.float32)
    y = y + lax.dot_general(h_ref[...], wr_ref[...], (((1,), (1,)), ((), ())),
                            preferred_element_type=jnp.float32)
    o_ref[...] = jnp.maximum(y + bl_ref[...], 0.0)


_tc_layer = pl.pallas_call(
    _layer_body,
    grid=(N // _BLK,),
    in_specs=[
        pl.BlockSpec((_BLK, D), lambda i: (i, 0)),
        pl.BlockSpec((_BLK, D), lambda i: (i, 0)),
        pl.BlockSpec((_BLK, D), lambda i: (i, 0)),
        pl.BlockSpec((_BLK, D), lambda i: (i, 0)),
        pl.BlockSpec((_BLK, D), lambda i: (i, 0)),
        pl.BlockSpec((D, D), lambda i: (0, 0)),
        pl.BlockSpec((1, D), lambda i: (0, 0)),
        pl.BlockSpec((D, D), lambda i: (0, 0)),
    ],
    out_specs=pl.BlockSpec((_BLK, D), lambda i: (i, 0)),
    out_shape=jax.ShapeDtypeStruct((N, D), jnp.float32),
)


def _final_body(p0_ref, p1_ref, c0_ref, c1_ref, h_ref, wl_ref, bl_ref, wr_ref,
                wp1_ref, bp1_ref, wp2_ref, bp2_ref, o_ref):
    cnt = c0_ref[:, 0:1] + c1_ref[:, 0:1]
    agg = (p0_ref[...] + p1_ref[...]) / jnp.maximum(cnt, 1.0)
    y = lax.dot_general(agg, wl_ref[...], (((1,), (1,)), ((), ())),
                        preferred_element_type=jnp.float32)
    y = y + lax.dot_general(h_ref[...], wr_ref[...], (((1,), (1,)), ((), ())),
                            preferred_element_type=jnp.float32)
    y = jnp.maximum(y + bl_ref[...], 0.0)
    y = jnp.maximum(
        lax.dot_general(y, wp1_ref[...], (((1,), (1,)), ((), ())),
                        preferred_element_type=jnp.float32) + bp1_ref[...], 0.0)
    o_ref[...] = lax.dot_general(
        y, wp2_ref[...], (((1,), (1,)), ((), ())),
        preferred_element_type=jnp.float32) + bp2_ref[...]


_tc_final = pl.pallas_call(
    _final_body,
    grid=(N // _BLK,),
    in_specs=[
        pl.BlockSpec((_BLK, D), lambda i: (i, 0)),
        pl.BlockSpec((_BLK, D), lambda i: (i, 0)),
        pl.BlockSpec((_BLK, D), lambda i: (i, 0)),
        pl.BlockSpec((_BLK, D), lambda i: (i, 0)),
        pl.BlockSpec((_BLK, D), lambda i: (i, 0)),
        pl.BlockSpec((D, D), lambda i: (0, 0)),
        pl.BlockSpec((1, D), lambda i: (0, 0)),
        pl.BlockSpec((D, D), lambda i: (0, 0)),
        pl.BlockSpec((D, D), lambda i: (0, 0)),
        pl.BlockSpec((1, D), lambda i: (0, 0)),
        pl.BlockSpec((D, D), lambda i: (0, 0)),
        pl.BlockSpec((1, D), lambda i: (0, 0)),
    ],
    out_specs=pl.BlockSpec((_BLK, D), lambda i: (i, 0)),
    out_shape=jax.ShapeDtypeStruct((N, D), jnp.float32),
)


def kernel(x, edge_attr, edge_index, W_pre, b_pre, Wl1, bl1, Wr1, We1, be1,
           Wl2, bl2, Wr2, We2, be2, Wp1, bp1, Wp2, bp2):
    del edge_attr, We1, be1, We2, be2
    src = edge_index[0]
    dst = edge_index[1]
    pad = E_PAD - E
    src_fill = jnp.arange(pad, dtype=jnp.int32) * 37 % N
    srcp = jnp.concatenate([src, src_fill]).reshape(NW, K, CH)
    dst_fill = N + (jnp.arange(pad, dtype=jnp.int32) % (N_PAD - N))
    dstp = jnp.concatenate([dst, dst_fill]).reshape(NW, K, CH)

    b_pre2 = b_pre.reshape(1, D)
    bl1_2 = bl1.reshape(1, D)
    bl2_2 = bl2.reshape(1, D)
    bp1_2 = bp1.reshape(1, D)
    bp2_2 = bp2.reshape(1, D)

    h0 = _tc_pre(x, W_pre, b_pre2)
    (cnts1,) = _sc_cnt()(dstp)
    h0b, cdep = lax.optimization_barrier((h0, cnts1))
    (parts1,) = _sc_segsum()(h0b, srcp, dstp)
    cnts1 = cdep
    c0 = cnts1[0, :N]
    c1 = cnts1[1, :N]
    h1 = _tc_layer(parts1[0, :N], parts1[1, :N], c0, c1, h0, Wl1, bl1_2, Wr1)
    (parts2,) = _sc_segsum()(h1, srcp, dstp)
    out = _tc_final(parts2[0, :N], parts2[1, :N], c0, c1, h1,
                    Wl2, bl2_2, Wr2, Wp1, bp1_2, Wp2, bp2_2)
    return out


# speedup vs baseline: 3.5827x; 1.3399x over previous
import functools

import jax
import jax.numpy as jnp
from jax import lax
from jax.experimental import pallas as pl
from jax.experimental.pallas import tpu as pltpu
from jax.experimental.pallas import tpu_sc as plsc

N = 10000
D = 128
E = 320000
NC = 2
NS = 16
NW = NC * NS
CH = 128
S = 2
R = 3
K = 84
E_PAD = NW * K * CH
N_PAD = 10112
ROWS_PT = N_PAD // NS
ZCH = 104


def _sc_cnt_body(dst_hbm, cnt_out, dst_v, ones_v, zrow_v, cnt_sh, sem):
    c = lax.axis_index("c")
    s = lax.axis_index("s")
    wid = c * NS + s
    zero16 = jnp.zeros((16,), jnp.float32)
    one16 = jnp.ones((16,), jnp.float32)

    def zrow_body(i, carry):
        for j in range(D // 16):
            zrow_v[i, pl.ds(j * 16, 16)] = zero16
        return carry

    lax.fori_loop(0, ZCH, zrow_body, 0)

    def ones_body(i, carry):
        for j in range(D // 16):
            ones_v[i, pl.ds(j * 16, 16)] = one16
        return carry

    lax.fori_loop(0, CH, ones_body, 0)

    base = s * ROWS_PT

    for z in range(ROWS_PT // ZCH):
        pltpu.sync_copy(zrow_v, cnt_sh.at[pl.ds(base + z * ZCH, ZCH)])
    rem = ROWS_PT - (ROWS_PT // ZCH) * ZCH
    if rem:
        pltpu.sync_copy(zrow_v.at[pl.ds(0, rem)],
                        cnt_sh.at[pl.ds(base + ROWS_PT - rem, rem)])
    pltpu.sync_copy(dst_hbm.at[wid], dst_v)
    plsc.subcore_barrier()

    def chunk_body(j, carry):
        pltpu.sync_copy(ones_v, cnt_sh.at[dst_v.at[j]], add=True)
        return carry

    lax.fori_loop(0, K, chunk_body, 0)
    plsc.subcore_barrier()

    pltpu.sync_copy(cnt_sh.at[pl.ds(base, ROWS_PT)],
                    cnt_out.at[c, pl.ds(base, ROWS_PT)])


@functools.cache
def _sc_cnt():
    mesh = plsc.VectorSubcoreMesh(core_axis_name="c", subcore_axis_name="s",
                                  num_cores=NC, num_subcores=NS)
    return pl.kernel(
        _sc_cnt_body,
        out_type=(jax.ShapeDtypeStruct((NC, N_PAD, D), jnp.float32),),
        mesh=mesh,
        scratch_types=[
            pltpu.VMEM((K, CH), jnp.int32),
            pltpu.VMEM((CH, D), jnp.float32),
            pltpu.VMEM((ZCH, D), jnp.float32),
            pltpu.VMEM_SHARED((N_PAD, D), jnp.float32),
            pltpu.SemaphoreType.DMA,
        ],
    )


def _sc_body(h_hbm, src_hbm, dst_hbm, parts_out,
             sring_v, dring_v, rows_v, zrow_v, acc_sh, isems, dsems,
             gsems, ssems):
    c = lax.axis_index("c")
    s = lax.axis_index("s")
    wid = c * NS + s
    zero16 = jnp.zeros((16,), jnp.float32)

    def fire_i(jj, r):
        pltpu.async_copy(src_hbm.at[wid, jj], sring_v.at[r], isems[r])

    def wait_i(jj, r):
        pltpu.make_async_copy(src_hbm.at[wid, jj], sring_v.at[r],
                              isems[r]).wait()

    def fire_di(jj, r):
        pltpu.async_copy(dst_hbm.at[wid, jj], dring_v.at[r], dsems[r])

    def wait_di(jj, r):
        pltpu.make_async_copy(dst_hbm.at[wid, jj], dring_v.at[r],
                              dsems[r]).wait()

    def fire_g(r, b):
        pltpu.async_copy(h_hbm.at[sring_v.at[r]], rows_v.at[b], gsems[b])

    def wait_g(r, b):
        pltpu.make_async_copy(h_hbm.at[sring_v.at[r]], rows_v.at[b],
                              gsems[b]).wait()

    def fire_s(r, b):
        pltpu.async_copy(rows_v.at[b], acc_sh.at[dring_v.at[r]], ssems[b],
                         add=True)

    def wait_s(r, b):
        pltpu.make_async_copy(rows_v.at[b], acc_sh.at[dring_v.at[r]],
                              ssems[b]).wait()

    def zrow_body(i, carry):
        for j in range(D // 16):
            zrow_v[i, pl.ds(j * 16, 16)] = zero16
        return carry

    lax.fori_loop(0, ZCH, zrow_body, 0)

    for r in range(R):
        fire_i(r, r)
        fire_di(r, r)

    base = s * ROWS_PT
    for z in range(ROWS_PT // ZCH):
        pltpu.sync_copy(zrow_v, acc_sh.at[pl.ds(base + z * ZCH, ZCH)])
    rem = ROWS_PT - (ROWS_PT // ZCH) * ZCH
    if rem:
        pltpu.sync_copy(zrow_v.at[pl.ds(0, rem)],
                        acc_sh.at[pl.ds(base + ROWS_PT - rem, rem)])
    wait_i(0, 0)
    fire_g(0, 0)
    plsc.subcore_barrier()

    def emit_step(j, off, do_wait_s=True, do_fire_g=True, do_fire_i=True,
                  do_fire_di=True):
        b0 = off % S
        b1 = (off + 1) % S
        r0 = off % R
        r1 = (off + 1) % R
        r2 = (off + 2) % R
        if do_wait_s:
            wait_s(r2, b1)
            if do_fire_di:
                fire_di(j + 2, r2)
        if do_fire_g:
            wait_i(j + 1, r1)
            fire_g(r1, b1)
        wait_g(r0, b0)
        if do_fire_i:
            fire_i(j + R, r0)
        wait_di(j, r0)
        fire_s(r0, b0)

    emit_step(0, 0, do_wait_s=False)
    for off in range(1, 6):
        emit_step(off, off)

    def super_body(g, carry):
        j0 = g * 6
        for off in range(6):
            emit_step(j0 + off, off)
        return carry

    lax.fori_loop(1, K // 6 - 1, super_body, 0)

    j0 = K - 6
    emit_step(j0 + 0, 0)
    emit_step(j0 + 1, 1)
    emit_step(j0 + 2, 2)
    emit_step(j0 + 3, 3, do_fire_i=False)
    emit_step(j0 + 4, 4, do_fire_i=False, do_fire_di=False)
    emit_step(j0 + 5, 5, do_fire_g=False, do_fire_i=False, do_fire_di=False)
    wait_s((K - 1) % R, (K - 1) % S)
    plsc.subcore_barrier()

    pltpu.sync_copy(acc_sh.at[pl.ds(base, ROWS_PT)],
                    parts_out.at[c, pl.ds(base, ROWS_PT)])


@functools.cache
def _sc_segsum():
    mesh = plsc.VectorSubcoreMesh(core_axis_name="c", subcore_axis_name="s",
                                  num_cores=NC, num_subcores=NS)
    return pl.kernel(
        _sc_body,
        out_type=(jax.ShapeDtypeStruct((NC, N_PAD, D), jnp.float32),),
        mesh=mesh,
        scratch_types=[
            pltpu.VMEM((R, CH), jnp.int32),
            pltpu.VMEM((R, CH), jnp.int32),
            pltpu.VMEM((S, CH, D), jnp.float32),
            pltpu.VMEM((ZCH, D), jnp.float32),
            pltpu.VMEM_SHARED((N_PAD, D), jnp.float32),
            [pltpu.SemaphoreType.DMA] * R,
            [pltpu.SemaphoreType.DMA] * R,
            [pltpu.SemaphoreType.DMA] * S,
            [pltpu.SemaphoreType.DMA] * S,
        ],
    )


_BLK = 2000


def _pre_body(x_ref, w_ref, b_ref, o_ref):
    o_ref[...] = lax.dot_general(
        x_ref[...], w_ref[...], (((1,), (1,)), ((), ())),
        preferred_element_type=jnp.float32) + b_ref[...]


_tc_pre = pl.pallas_call(
    _pre_body,
    grid=(N // _BLK,),
    in_specs=[
        pl.BlockSpec((_BLK, D), lambda i: (i, 0)),
        pl.BlockSpec((D, D), lambda i: (0, 0)),
        pl.BlockSpec((1, D), lambda i: (0, 0)),
    ],
    out_specs=pl.BlockSpec((_BLK, D), lambda i: (i, 0)),
    out_shape=jax.ShapeDtypeStruct((N, D), jnp.float32),
)


def _layer_body(p0_ref, p1_ref, c0_ref, c1_ref, h_ref, wl_ref, bl_ref, wr_ref, o_ref):
    cnt = c0_ref[:, 0:1] + c1_ref[:, 0:1]
    agg = (p0_ref[...] + p1_ref[...]) / jnp.maximum(cnt, 1.0)
    y = lax.dot_general(agg, wl_ref[...], (((1,), (1,)), ((), ())),
                        preferred_element_type=jnp.float32)
    y = y + lax.dot_general(h_ref[...], wr_ref[...], (((1,), (1,)), ((), ())),
                            preferred_element_type=jnp.float32)
    o_ref[...] = jnp.maximum(y + bl_ref[...], 0.0)


_tc_layer = pl.pallas_call(
    _layer_body,
    grid=(N // _BLK,),
    in_specs=[
        pl.BlockSpec((_BLK, D), lambda i: (i, 0)),
        pl.BlockSpec((_BLK, D), lambda i: (i, 0)),
        pl.BlockSpec((_BLK, D), lambda i: (i, 0)),
        pl.BlockSpec((_BLK, D), lambda i: (i, 0)),
        pl.BlockSpec((_BLK, D), lambda i: (i, 0)),
        pl.BlockSpec((D, D), lambda i: (0, 0)),
        pl.BlockSpec((1, D), lambda i: (0, 0)),
        pl.BlockSpec((D, D), lambda i: (0, 0)),
    ],
    out_specs=pl.BlockSpec((_BLK, D), lambda i: (i, 0)),
    out_shape=jax.ShapeDtypeStruct((N, D), jnp.float32),
)


def _final_body(p0_ref, p1_ref, c0_ref, c1_ref, h_ref, wl_ref, bl_ref, wr_ref,
                wp1_ref, bp1_ref, wp2_ref, bp2_ref, o_ref):
    cnt = c0_ref[:, 0:1] + c1_ref[:, 0:1]
    agg = (p0_ref[...] + p1_ref[...]) / jnp.maximum(cnt, 1.0)
    y = lax.dot_general(agg, wl_ref[...], (((1,), (1,)), ((), ())),
                        preferred_element_type=jnp.float32)
    y = y + lax.dot_general(h_ref[...], wr_ref[...], (((1,), (1,)), ((), ())),
                            preferred_element_type=jnp.float32)
    y = jnp.maximum(y + bl_ref[...], 0.0)
    y = jnp.maximum(
        lax.dot_general(y, wp1_ref[...], (((1,), (1,)), ((), ())),
                        preferred_element_type=jnp.float32) + bp1_ref[...], 0.0)
    o_ref[...] = lax.dot_general(
        y, wp2_ref[...], (((1,), (1,)), ((), ())),
        preferred_element_type=jnp.float32) + bp2_ref[...]


_tc_final = pl.pallas_call(
    _final_body,
    grid=(N // _BLK,),
    in_specs=[
        pl.BlockSpec((_BLK, D), lambda i: (i, 0)),
        pl.BlockSpec((_BLK, D), lambda i: (i, 0)),
        pl.BlockSpec((_BLK, D), lambda i: (i, 0)),
        pl.BlockSpec((_BLK, D), lambda i: (i, 0)),
        pl.BlockSpec((_BLK, D), lambda i: (i, 0)),
        pl.BlockSpec((D, D), lambda i: (0, 0)),
        pl.BlockSpec((1, D), lambda i: (0, 0)),
        pl.BlockSpec((D, D), lambda i: (0, 0)),
        pl.BlockSpec((D, D), lambda i: (0, 0)),
        pl.BlockSpec((1, D), lambda i: (0, 0)),
        pl.BlockSpec((D, D), lambda i: (0, 0)),
        pl.BlockSpec((1, D), lambda i: (0, 0)),
    ],
    out_specs=pl.BlockSpec((_BLK, D), lambda i: (i, 0)),
    out_shape=jax.ShapeDtypeStruct((N, D), jnp.float32),
)


def kernel(x, edge_attr, edge_index, W_pre, b_pre, Wl1, bl1, Wr1, We1, be1,
           Wl2, bl2, Wr2, We2, be2, Wp1, bp1, Wp2, bp2):
    del edge_attr, We1, be1, We2, be2
    src = edge_index[0]
    dst = edge_index[1]
    pad = E_PAD - E
    src_fill = jnp.arange(pad, dtype=jnp.int32) * 37 % N
    srcp = jnp.concatenate([src, src_fill]).reshape(NW, K, CH)
    dst_fill = N + (jnp.arange(pad, dtype=jnp.int32) % (N_PAD - N))
    dstp = jnp.concatenate([dst, dst_fill]).reshape(NW, K, CH)

    b_pre2 = b_pre.reshape(1, D)
    bl1_2 = bl1.reshape(1, D)
    bl2_2 = bl2.reshape(1, D)
    bp1_2 = bp1.reshape(1, D)
    bp2_2 = bp2.reshape(1, D)

    h0 = _tc_pre(x, W_pre, b_pre2)
    (cnts1,) = _sc_cnt()(dstp)
    h0b, cdep = lax.optimization_barrier((h0, cnts1))
    (parts1,) = _sc_segsum()(h0b, srcp, dstp)
    cnts1 = cdep
    c0 = cnts1[0, :N]
    c1 = cnts1[1, :N]
    h1 = _tc_layer(parts1[0, :N], parts1[1, :N], c0, c1, h0, Wl1, bl1_2, Wr1)
    (parts2,) = _sc_segsum()(h1, srcp, dstp)
    out = _tc_final(parts2[0, :N], parts2[1, :N], c0, c1, h1,
                    Wl2, bl2_2, Wr2, Wp1, bp1_2, Wp2, bp2_2)
    return out
